# baseline (speedup 1.0000x reference)
import jax
import jax.numpy as jnp
from jax.experimental import pallas as pl

_RB = 1024


def _make_body(S):
    nsb = S // _RB

    def _add_body(x_ref, pe_ref, o_ref):
        i = pl.program_id(0) % nsb
        o_ref[...] = x_ref[...] + pe_ref[pl.ds(i * _RB, _RB), :]

    return _add_body


def kernel(inputs, pos_embed):
    B, S, D = inputs.shape
    x2d = inputs.reshape(B * S, D)
    out = pl.pallas_call(
        _make_body(S),
        grid=(B * S // _RB,),
        in_specs=[
            pl.BlockSpec((_RB, D), lambda i: (i, 0)),
            pl.BlockSpec((S, D), lambda i: (0, 0)),
        ],
        out_specs=pl.BlockSpec((_RB, D), lambda i: (i, 0)),
        out_shape=jax.ShapeDtypeStruct((B * S, D), inputs.dtype),
    )(x2d, pos_embed)
    return out.reshape(B, S, D)


# speedup vs baseline: 1.0682x; 1.0682x over previous
import jax
import jax.numpy as jnp
from jax.experimental import pallas as pl


def _add_body(x_ref, pe_ref, o_ref):
    o_ref[...] = x_ref[...] + pe_ref[...]


def kernel(inputs, pos_embed):
    B, S, D = inputs.shape
    x2d = inputs.reshape(B * S, D)
    out = pl.pallas_call(
        _add_body,
        grid=(B,),
        in_specs=[
            pl.BlockSpec((S, D), lambda i: (i, 0)),
            pl.BlockSpec((S, D), lambda i: (0, 0)),
        ],
        out_specs=pl.BlockSpec((S, D), lambda i: (i, 0)),
        out_shape=jax.ShapeDtypeStruct((B * S, D), inputs.dtype),
    )(x2d, pos_embed)
    return out.reshape(B, S, D)
